# baseline (speedup 1.0000x reference)
import functools

import jax
import jax.numpy as jnp
from jax import lax
from jax.experimental import pallas as pl
from jax.experimental.pallas import tpu as pltpu
from jax.experimental.pallas import tpu_sc as plsc

HIDDEN = 64
EPS = 1e-12
SB = 800
CP = 80


def _make_sc_gather(pairs):
    info = plsc.get_sparse_core_info()
    nc, nw = info.num_cores, info.num_cores * info.num_subcores
    rpw = pairs // nw
    nchunk = rpw // CP
    mesh = plsc.VectorSubcoreMesh(core_axis_name="c", subcore_axis_name="s")

    @functools.partial(
        pl.kernel,
        mesh=mesh,
        compiler_params=pltpu.CompilerParams(use_tc_tiling_on_sc=False),
        out_type=jax.ShapeDtypeStruct((pairs, 2 * HIDDEN), jnp.float32),
        scratch_types=[
            pltpu.VMEM((CP,), jnp.int32),
            pltpu.VMEM((CP,), jnp.int32),
            pltpu.VMEM((CP, HIDDEN), jnp.float32),
            pltpu.VMEM((CP, HIDDEN), jnp.float32),
            pltpu.SemaphoreType.DMA,
            pltpu.SemaphoreType.DMA,
        ],
    )
    def sc_gather(idx_hbm, tab_hbm, out_hbm, idxl_v, idxr_v, rl_v, rr_v,
                  seml, semr):
        w = lax.axis_index("s") * nc + lax.axis_index("c")

        def body(c, carry):
            p = w * rpw + c * CP
            sb = p // SB
            lb = sb * (2 * SB) + (p - sb * SB)
            pltpu.sync_copy(idx_hbm.at[pl.ds(lb, CP)], idxl_v)
            pltpu.sync_copy(idx_hbm.at[pl.ds(lb + SB, CP)], idxr_v)
            cl = pltpu.async_copy(tab_hbm.at[idxl_v], rl_v, seml)
            cr = pltpu.async_copy(tab_hbm.at[idxr_v], rr_v, semr)
            cl.wait()
            cr.wait()
            pltpu.sync_copy(rl_v, out_hbm.at[pl.ds(p, CP), pl.ds(0, HIDDEN)])
            pltpu.sync_copy(rr_v,
                            out_hbm.at[pl.ds(p, CP), pl.ds(HIDDEN, HIDDEN)])
            return carry

        lax.fori_loop(0, nchunk, body, 0)

    return sc_gather


def _ln_compute(aid_ref, rows_ref, atab2_ref, bd_ref, g_ref, b_ref, out_ref):
    x = rows_ref[...]
    ids = aid_ref[0, 0, :]
    oh = [
        (ids[h * SB:(h + 1) * SB][:, None] ==
         lax.broadcasted_iota(jnp.int32, (SB, 128), 1)).astype(jnp.float32)
        for h in range(2)
    ]
    age = (jnp.dot(oh[0], atab2_ref[0], preferred_element_type=jnp.float32) +
           jnp.dot(oh[1], atab2_ref[1], preferred_element_type=jnp.float32))
    xh = x + age
    bd = bd_ref[...]
    u = jnp.dot(xh, bd, preferred_element_type=jnp.float32)
    d = xh - u
    s = jnp.dot(d * d, bd, preferred_element_type=jnp.float32)
    y = g_ref[...] * (d * lax.rsqrt(s + EPS)) + b_ref[...]
    z = jnp.concatenate([y[:, :HIDDEN], y[:, HIDDEN:]], axis=0)
    out_ref[...] = z.reshape(out_ref.shape)


def _ln_body_rest(aid_ref, rows_ref, atab2_ref, bd_ref, g_ref, b_ref, buf_ref,
                  out_ref):
    del buf_ref
    _ln_compute(aid_ref, rows_ref, atab2_ref, bd_ref, g_ref, b_ref, out_ref)


SEQ = 200
MB = (2 * SB) // SEQ


def _make_tc_ln(b, rs, sl):
    grid = rs // (2 * SB)
    base = sl * grid
    in_specs = [
        pl.BlockSpec((1, 1, 2 * SB), lambda i: (base + i, 0, 0)),
        pl.BlockSpec((SB, 2 * HIDDEN), lambda i: (i, 0)),
        pl.BlockSpec((2, 128, 128), lambda i: (0, 0, 0)),
        pl.BlockSpec((128, 128), lambda i: (0, 0)),
        pl.BlockSpec((1, 2 * HIDDEN), lambda i: (0, 0)),
        pl.BlockSpec((1, 2 * HIDDEN), lambda i: (0, 0)),
    ]
    kwargs = {}
    body = _ln_compute
    if sl > 0:
        in_specs.append(pl.BlockSpec(memory_space=pl.ANY))
        kwargs["input_output_aliases"] = {6: 0}
        body = _ln_body_rest
    return pl.pallas_call(
        body,
        grid=(grid,),
        in_specs=in_specs,
        out_specs=pl.BlockSpec((MB, SEQ, HIDDEN), lambda i: (base + i, 0, 0)),
        out_shape=jax.ShapeDtypeStruct((b, SEQ, HIDDEN), jnp.float32),
        **kwargs,
    )


NSLICE = 4


def kernel(word_ids, age_ids, word_table, age_table, gamma, beta):
    b, l = word_ids.shape
    rows = b * l
    rs = rows // NSLICE
    wids = word_ids.reshape(NSLICE, rs)
    atab2 = jnp.zeros((2, 128, 128), jnp.float32)
    atab2 = atab2.at[0, :, :HIDDEN].set(age_table)
    atab2 = atab2.at[1, :, HIDDEN:].set(age_table)
    bd = jnp.zeros((128, 128), jnp.float32)
    bd = bd.at[:HIDDEN, :HIDDEN].set(1.0 / HIDDEN)
    bd = bd.at[HIDDEN:, HIDDEN:].set(1.0 / HIDDEN)
    g2 = jnp.concatenate([gamma, gamma]).reshape(1, 2 * HIDDEN)
    b2 = jnp.concatenate([beta, beta]).reshape(1, 2 * HIDDEN)
    aids = age_ids.reshape(rows // (2 * SB), 1, 2 * SB)
    sc_gather = _make_sc_gather(rs // 2)
    buf = None
    for s in range(NSLICE):
        packed = sc_gather(wids[s], word_table)
        args = (aids, packed, atab2, bd, g2, b2)
        if s > 0:
            args += (buf,)
        buf = _make_tc_ln(b, rs, s)(*args)
    return buf


# speedup vs baseline: 1.1440x; 1.1440x over previous
import functools

import jax
import jax.numpy as jnp
from jax import lax
from jax.experimental import pallas as pl
from jax.experimental.pallas import tpu as pltpu
from jax.experimental.pallas import tpu_sc as plsc

HIDDEN = 64
EPS = 1e-12
SB = 800
CP = 80


def _make_sc_gather(pairs):
    info = plsc.get_sparse_core_info()
    nc, nw = info.num_cores, info.num_cores * info.num_subcores
    rpw = pairs // nw
    nchunk = rpw // CP
    mesh = plsc.VectorSubcoreMesh(core_axis_name="c", subcore_axis_name="s")

    @functools.partial(
        pl.kernel,
        mesh=mesh,
        compiler_params=pltpu.CompilerParams(use_tc_tiling_on_sc=False),
        out_type=jax.ShapeDtypeStruct((pairs, 2 * HIDDEN), jnp.float32),
        scratch_types=[
            pltpu.VMEM((CP,), jnp.int32),
            pltpu.VMEM((CP,), jnp.int32),
            pltpu.VMEM((CP, HIDDEN), jnp.float32),
            pltpu.VMEM((CP, HIDDEN), jnp.float32),
            pltpu.SemaphoreType.DMA,
            pltpu.SemaphoreType.DMA,
        ],
    )
    def sc_gather(idx_hbm, tab_hbm, out_hbm, idxl_v, idxr_v, rl_v, rr_v,
                  seml, semr):
        w = lax.axis_index("s") * nc + lax.axis_index("c")

        def body(c, carry):
            p = w * rpw + c * CP
            sb = p // SB
            lb = sb * (2 * SB) + (p - sb * SB)
            pltpu.sync_copy(idx_hbm.at[pl.ds(lb, CP)], idxl_v)
            pltpu.sync_copy(idx_hbm.at[pl.ds(lb + SB, CP)], idxr_v)
            cl = pltpu.async_copy(tab_hbm.at[idxl_v], rl_v, seml)
            cr = pltpu.async_copy(tab_hbm.at[idxr_v], rr_v, semr)
            cl.wait()
            cr.wait()
            pltpu.sync_copy(rl_v, out_hbm.at[pl.ds(p, CP), pl.ds(0, HIDDEN)])
            pltpu.sync_copy(rr_v,
                            out_hbm.at[pl.ds(p, CP), pl.ds(HIDDEN, HIDDEN)])
            return carry

        lax.fori_loop(0, nchunk, body, 0)

    return sc_gather


def _ln_compute(aid_ref, rows_ref, atab2_ref, bd_ref, g_ref, b_ref, out_ref):
    x = rows_ref[...]
    ids = aid_ref[0, 0, :]
    oh = [
        (ids[h * SB:(h + 1) * SB][:, None] ==
         lax.broadcasted_iota(jnp.int32, (SB, 128), 1)).astype(jnp.float32)
        for h in range(2)
    ]
    age = (jnp.dot(oh[0], atab2_ref[0], preferred_element_type=jnp.float32) +
           jnp.dot(oh[1], atab2_ref[1], preferred_element_type=jnp.float32))
    xh = x + age
    bd = bd_ref[...]
    u = jnp.dot(xh, bd, preferred_element_type=jnp.float32)
    d = xh - u
    s = jnp.dot(d * d, bd, preferred_element_type=jnp.float32)
    y = g_ref[...] * (d * lax.rsqrt(s + EPS)) + b_ref[...]
    out_ref[...] = jnp.concatenate([y[:, :HIDDEN], y[:, HIDDEN:]], axis=0)


def _ln_body_rest(aid_ref, rows_ref, atab2_ref, bd_ref, g_ref, b_ref, buf_ref,
                  out_ref):
    del buf_ref
    _ln_compute(aid_ref, rows_ref, atab2_ref, bd_ref, g_ref, b_ref, out_ref)


def _make_tc_ln(rows, rs, sl):
    grid = rs // (2 * SB)
    base = sl * grid
    in_specs = [
        pl.BlockSpec((1, 1, 2 * SB), lambda i: (base + i, 0, 0)),
        pl.BlockSpec((SB, 2 * HIDDEN), lambda i: (i, 0)),
        pl.BlockSpec((2, 128, 128), lambda i: (0, 0, 0)),
        pl.BlockSpec((128, 128), lambda i: (0, 0)),
        pl.BlockSpec((1, 2 * HIDDEN), lambda i: (0, 0)),
        pl.BlockSpec((1, 2 * HIDDEN), lambda i: (0, 0)),
    ]
    kwargs = {}
    body = _ln_compute
    if sl > 0:
        in_specs.append(pl.BlockSpec(memory_space=pl.ANY))
        kwargs["input_output_aliases"] = {6: 0}
        body = _ln_body_rest
    return pl.pallas_call(
        body,
        grid=(grid,),
        in_specs=in_specs,
        out_specs=pl.BlockSpec((2 * SB, HIDDEN), lambda i: (base + i, 0)),
        out_shape=jax.ShapeDtypeStruct((rows, HIDDEN), jnp.float32),
        **kwargs,
    )


NSLICE = 8


def kernel(word_ids, age_ids, word_table, age_table, gamma, beta):
    b, l = word_ids.shape
    rows = b * l
    rs = rows // NSLICE
    wids = word_ids.reshape(NSLICE, rs)
    atab2 = jnp.zeros((2, 128, 128), jnp.float32)
    atab2 = atab2.at[0, :, :HIDDEN].set(age_table)
    atab2 = atab2.at[1, :, HIDDEN:].set(age_table)
    bd = jnp.zeros((128, 128), jnp.float32)
    bd = bd.at[:HIDDEN, :HIDDEN].set(1.0 / HIDDEN)
    bd = bd.at[HIDDEN:, HIDDEN:].set(1.0 / HIDDEN)
    g2 = jnp.concatenate([gamma, gamma]).reshape(1, 2 * HIDDEN)
    b2 = jnp.concatenate([beta, beta]).reshape(1, 2 * HIDDEN)
    aids = age_ids.reshape(rows // (2 * SB), 1, 2 * SB)
    sc_gather = _make_sc_gather(rs // 2)
    buf = None
    for s in range(NSLICE):
        packed = sc_gather(wids[s], word_table)
        args = (aids, packed, atab2, bd, g2, b2)
        if s > 0:
            args += (buf,)
        buf = _make_tc_ln(rows, rs, s)(*args)
    return buf.reshape(b, l, HIDDEN)
